# baseline (speedup 1.0000x reference)
import jax
import jax.numpy as jnp
import numpy as np
from jax.experimental import pallas as pl
from jax.experimental.pallas import tpu as pltpu

LATENT = 128
HIDDEN = 128
N_NODES = 256
HEADS = 4
OUTC = HIDDEN // HEADS
BATCH = 16
NEG_SLOPE = 0.2

NODE_BLK = 16
NB = N_NODES // NODE_BLK
_PREC = jax.lax.Precision.HIGHEST


def _np_threefry_uniform(seed, shape):
    def rotl(x, d):
        return (x << np.uint32(d)) | (x >> np.uint32(32 - d))

    n = int(np.prod(shape))
    idx = np.arange(n, dtype=np.uint64)
    c1 = (idx >> np.uint64(32)).astype(np.uint32)
    c2 = (idx & np.uint64(0xFFFFFFFF)).astype(np.uint32)
    k1 = np.uint32(seed >> 32)
    k2 = np.uint32(seed & 0xFFFFFFFF)
    rot = ([13, 15, 26, 6], [17, 29, 16, 24])
    ks = (k1, k2, k1 ^ k2 ^ np.uint32(0x1BD11BDA))
    x = [c1 + ks[0], c2 + ks[1]]
    for i in range(5):
        for r in rot[i % 2]:
            x[0] = x[0] + x[1]
            x[1] = rotl(x[1], r) ^ x[0]
        x[0] = x[0] + ks[(i + 1) % 3]
        x[1] = x[1] + ks[(i + 2) % 3] + np.uint32(i + 1)
    bits = x[0] ^ x[1]
    float_bits = (bits >> np.uint32(9)) | np.uint32(0x3F800000)
    floats = float_bits.view(np.float32) - np.float32(1.0)
    tiny = np.float32(np.finfo(np.float32).tiny)
    span = np.float32(np.float32(1.0) - tiny)
    return np.maximum(tiny, floats * span + tiny).reshape(shape)


_GUMBEL_U = _np_threefry_uniform(42, (BATCH, N_NODES, N_NODES))


def _fused_kernel(x_ref, w0_ref, b0_ref, w1_ref, b1_ref, w2_ref, b2_ref,
                  gw_ref, p_ref, gb_ref, wl_ref, bl_ref, g_ref,
                  out_ref, xh_scr):
    i = pl.program_id(0)

    @pl.when(i < NB)
    def _stage_a():
        h = jax.lax.dot_general(x_ref[...], w0_ref[...],
                                (((1,), (1,)), ((), ())), precision=_PREC)
        h = jnp.maximum(h + b0_ref[...], 0.0)
        h = jax.lax.dot_general(h, w1_ref[...], (((1,), (1,)), ((), ())),
                                precision=_PREC)
        h = jnp.maximum(h + b1_ref[...], 0.0)
        h2 = jax.lax.dot_general(h, w2_ref[...], (((1,), (2,)), ((), ())),
                                 precision=_PREC)
        h2 = h2 + b2_ref[...][None, :, :]
        xh = jax.lax.dot_general(h2, gw_ref[...], (((2,), (1,)), ((), ())),
                                 precision=_PREC)
        xh_scr[i] = xh

    @pl.when(i >= NB)
    def _stage_b():
        b = i - NB
        xh = jnp.concatenate([xh_scr[nb, b] for nb in range(NB)],
                             axis=0)
        sc = jnp.dot(xh, p_ref[...], precision=_PREC)
        scT = jax.lax.dot_general(p_ref[...], xh, (((0,), (1,)), ((), ())),
                                  precision=_PREC)
        HN = N_NODES // 2
        ii = jax.lax.broadcasted_iota(jnp.int32, (HN, HN), 0)
        jj = jax.lax.broadcasted_iota(jnp.int32, (HN, HN), 1)
        tri = jnp.where(ii <= jj, 0.0, -jnp.inf)
        mask_lo = jnp.concatenate(
            [tri, jnp.zeros((HN, HN), jnp.float32)], axis=1)
        ones_col = jnp.ones((N_NODES, 1), dtype=jnp.float32)
        aggs = []
        for h in range(HEADS):
            arow = scT[HEADS + h:HEADS + h + 1, :]
            sl = sc[:HN, h:h + 1] + arow
            sh = sc[HN:, h:h + 1] + arow[:, HN:]
            sl = jnp.where(sl >= 0.0, sl, NEG_SLOPE * sl) + mask_lo
            sh = jnp.where(sh >= 0.0, sh, NEG_SLOPE * sh) + tri
            cl = jnp.max(sl, axis=0, keepdims=True)
            ch = jnp.max(sh, axis=0, keepdims=True)
            cmax = jnp.concatenate(
                [cl[:, :HN], jnp.maximum(cl[:, HN:], ch)], axis=1)
            ex_lo = jnp.exp(sl - cmax)
            ex_hi = jnp.exp(sh - cmax[:, HN:])
            xh_h = jnp.concatenate(
                [xh[:, h * OUTC:(h + 1) * OUTC], ones_col],
                axis=1)
            m = jax.lax.dot_general(ex_lo, xh_h[:HN],
                                    (((0,), (0,)), ((), ())),
                                    precision=_PREC)
            mb = jax.lax.dot_general(ex_hi, xh_h[HN:],
                                     (((0,), (0,)), ((), ())),
                                     precision=_PREC)
            m = m + jnp.concatenate(
                [jnp.zeros((HN, OUTC + 1), jnp.float32), mb], axis=0)
            recip = 1.0 / (m[:, OUTC:OUTC + 1] + 1e-16)
            aggs.append(m[:, :OUTC] * recip)
        agg = jnp.concatenate(aggs, axis=1)
        out = agg + gb_ref[...]
        out = jnp.where(out > 0.0, out,
                        jnp.exp(jnp.minimum(out, 0.0)) - 1.0)
        logits = jax.lax.dot_general(out, wl_ref[...],
                                     (((1,), (1,)), ((), ())),
                                     precision=_PREC)
        z = logits + bl_ref[...] + g_ref[0]
        jjf = jax.lax.broadcasted_iota(jnp.int32, (N_NODES, N_NODES), 1)
        rmax = jnp.max(z, axis=1, keepdims=True)
        eq = z == rmax
        idx = jnp.min(jnp.where(eq, jjf, N_NODES), axis=1, keepdims=True)
        y = (jjf == idx).astype(jnp.float32)
        adj = jnp.minimum(y + y.T, 1.0)
        out_ref[0] = adj


@jax.jit
def kernel(x, W0, b0, W1, b1, W2, b2, gat_W, att_src, att_dst, gat_b, Wl, bl):
    B = x.shape[0]
    w2r = W2.reshape(N_NODES, HIDDEN, HIDDEN)
    b2r = b2.reshape(N_NODES, HIDDEN)

    eye = jnp.eye(HEADS, dtype=jnp.float32)
    p_src = (eye[:, None, :] * att_src[:, :, None]).reshape(HIDDEN, HEADS)
    p_dst = (eye[:, None, :] * att_dst[:, :, None]).reshape(HIDDEN, HEADS)
    P = jnp.concatenate([p_src, p_dst], axis=1)

    g = -jnp.log(-jnp.log(_GUMBEL_U))

    const = lambda i: (0, 0)
    adj = pl.pallas_call(
        _fused_kernel,
        grid=(NB + B,),
        in_specs=[
            pl.BlockSpec((B, LATENT), const),
            pl.BlockSpec((HIDDEN, LATENT), const),
            pl.BlockSpec((1, HIDDEN), const),
            pl.BlockSpec((HIDDEN, HIDDEN), const),
            pl.BlockSpec((1, HIDDEN), const),
            pl.BlockSpec((NODE_BLK, HIDDEN, HIDDEN),
                         lambda i: (jnp.minimum(i, NB - 1), 0, 0)),
            pl.BlockSpec((NODE_BLK, HIDDEN),
                         lambda i: (jnp.minimum(i, NB - 1), 0)),
            pl.BlockSpec((HIDDEN, HIDDEN), const),
            pl.BlockSpec((HIDDEN, 2 * HEADS), const),
            pl.BlockSpec((1, HIDDEN), const),
            pl.BlockSpec((N_NODES, HIDDEN), const),
            pl.BlockSpec((1, N_NODES), const),
            pl.BlockSpec((1, N_NODES, N_NODES),
                         lambda i: (jnp.maximum(i - NB, 0), 0, 0)),
        ],
        out_specs=pl.BlockSpec((1, N_NODES, N_NODES),
                               lambda i: (jnp.maximum(i - NB, 0), 0, 0)),
        out_shape=jax.ShapeDtypeStruct((B, N_NODES, N_NODES), jnp.float32),
        scratch_shapes=[pltpu.VMEM((NB, B, NODE_BLK, HIDDEN), jnp.float32)],
    )(x, W0, b0.reshape(1, -1), W1, b1.reshape(1, -1), w2r, b2r, gat_W,
      P, gat_b.reshape(1, -1), Wl, bl.reshape(1, -1), g)
    return adj


# speedup vs baseline: 1.1895x; 1.1895x over previous
import jax
import jax.numpy as jnp
import numpy as np
from jax.experimental import pallas as pl
from jax.experimental.pallas import tpu as pltpu

LATENT = 128
HIDDEN = 128
N_NODES = 256
HEADS = 4
OUTC = HIDDEN // HEADS
BATCH = 16
NEG_SLOPE = 0.2

NODE_BLK = 32
NB = N_NODES // NODE_BLK
GPB = 2
_PREC = jax.lax.Precision.HIGHEST


def _np_threefry_uniform(seed, shape):
    def rotl(x, d):
        return (x << np.uint32(d)) | (x >> np.uint32(32 - d))

    n = int(np.prod(shape))
    idx = np.arange(n, dtype=np.uint64)
    c1 = (idx >> np.uint64(32)).astype(np.uint32)
    c2 = (idx & np.uint64(0xFFFFFFFF)).astype(np.uint32)
    k1 = np.uint32(seed >> 32)
    k2 = np.uint32(seed & 0xFFFFFFFF)
    rot = ([13, 15, 26, 6], [17, 29, 16, 24])
    ks = (k1, k2, k1 ^ k2 ^ np.uint32(0x1BD11BDA))
    x = [c1 + ks[0], c2 + ks[1]]
    for i in range(5):
        for r in rot[i % 2]:
            x[0] = x[0] + x[1]
            x[1] = rotl(x[1], r) ^ x[0]
        x[0] = x[0] + ks[(i + 1) % 3]
        x[1] = x[1] + ks[(i + 2) % 3] + np.uint32(i + 1)
    bits = x[0] ^ x[1]
    float_bits = (bits >> np.uint32(9)) | np.uint32(0x3F800000)
    floats = float_bits.view(np.float32) - np.float32(1.0)
    tiny = np.float32(np.finfo(np.float32).tiny)
    span = np.float32(np.float32(1.0) - tiny)
    return np.maximum(tiny, floats * span + tiny).reshape(shape)


_GUMBEL_U = _np_threefry_uniform(42, (BATCH, N_NODES, N_NODES))


def _fused_kernel(x_ref, w0_ref, b0_ref, w1_ref, b1_ref, w2_ref, b2_ref,
                  gw_ref, p_ref, gb_ref, wl_ref, bl_ref, g_ref,
                  out_ref, xh_scr):
    i = pl.program_id(0)

    @pl.when(i < NB)
    def _stage_a():
        h = jax.lax.dot_general(x_ref[...], w0_ref[...],
                                (((1,), (1,)), ((), ())), precision=_PREC)
        h = jnp.maximum(h + b0_ref[...], 0.0)
        h = jax.lax.dot_general(h, w1_ref[...], (((1,), (1,)), ((), ())),
                                precision=_PREC)
        h = jnp.maximum(h + b1_ref[...], 0.0)
        h2 = jax.lax.dot_general(h, w2_ref[...], (((1,), (2,)), ((), ())),
                                 precision=_PREC)
        h2 = h2 + b2_ref[...][None, :, :]
        xh = jax.lax.dot_general(h2, gw_ref[...], (((2,), (1,)), ((), ())),
                                 precision=_PREC)
        xh_scr[i] = xh

    @pl.when(i >= NB)
    def _stage_b():
        for gi in range(GPB):
            _one_graph(i - NB, gi, p_ref, gb_ref, wl_ref, bl_ref, g_ref,
                       out_ref, xh_scr)


def _one_graph(b, gi, p_ref, gb_ref, wl_ref, bl_ref, g_ref, out_ref, xh_scr):
    if True:
        xh = jnp.concatenate([xh_scr[nb, b * GPB + gi] for nb in range(NB)],
                             axis=0)
        sc = jnp.dot(xh, p_ref[...], precision=_PREC)
        scT = jax.lax.dot_general(p_ref[...], xh, (((0,), (1,)), ((), ())),
                                  precision=_PREC)
        HN = N_NODES // 2
        ii = jax.lax.broadcasted_iota(jnp.int32, (HN, HN), 0)
        jj = jax.lax.broadcasted_iota(jnp.int32, (HN, HN), 1)
        tri = jnp.where(ii <= jj, 0.0, -jnp.inf)
        mask_lo = jnp.concatenate(
            [tri, jnp.zeros((HN, HN), jnp.float32)], axis=1)
        ones_col = jnp.ones((N_NODES, 1), dtype=jnp.float32)
        aggs = []
        for h in range(HEADS):
            arow = scT[HEADS + h:HEADS + h + 1, :]
            sl = sc[:HN, h:h + 1] + arow
            sh = sc[HN:, h:h + 1] + arow[:, HN:]
            sl = jnp.where(sl >= 0.0, sl, NEG_SLOPE * sl) + mask_lo
            sh = jnp.where(sh >= 0.0, sh, NEG_SLOPE * sh) + tri
            cl = jnp.max(sl, axis=0, keepdims=True)
            ch = jnp.max(sh, axis=0, keepdims=True)
            cmax = jnp.concatenate(
                [cl[:, :HN], jnp.maximum(cl[:, HN:], ch)], axis=1)
            ex_lo = jnp.exp(sl - cmax)
            ex_hi = jnp.exp(sh - cmax[:, HN:])
            xh_h = jnp.concatenate(
                [xh[:, h * OUTC:(h + 1) * OUTC], ones_col],
                axis=1)
            m = jax.lax.dot_general(ex_lo, xh_h[:HN],
                                    (((0,), (0,)), ((), ())),
                                    precision=_PREC)
            mb = jax.lax.dot_general(ex_hi, xh_h[HN:],
                                     (((0,), (0,)), ((), ())),
                                     precision=_PREC)
            m = m + jnp.concatenate(
                [jnp.zeros((HN, OUTC + 1), jnp.float32), mb], axis=0)
            recip = 1.0 / (m[:, OUTC:OUTC + 1] + 1e-16)
            aggs.append(m[:, :OUTC] * recip)
        agg = jnp.concatenate(aggs, axis=1)
        out = agg + gb_ref[...]
        out = jnp.where(out > 0.0, out,
                        jnp.exp(jnp.minimum(out, 0.0)) - 1.0)
        logits = jax.lax.dot_general(out, wl_ref[...],
                                     (((1,), (1,)), ((), ())),
                                     precision=_PREC)
        z = logits + bl_ref[...] + g_ref[gi]
        jjf = jax.lax.broadcasted_iota(jnp.int32, (N_NODES, N_NODES), 1)
        rmax = jnp.max(z, axis=1, keepdims=True)
        eq = z == rmax
        idx = jnp.min(jnp.where(eq, jjf, N_NODES), axis=1, keepdims=True)
        y = (jjf == idx).astype(jnp.float32)
        adj = jnp.minimum(y + y.T, 1.0)
        out_ref[gi] = adj


@jax.jit
def kernel(x, W0, b0, W1, b1, W2, b2, gat_W, att_src, att_dst, gat_b, Wl, bl):
    B = x.shape[0]
    w2r = W2.reshape(N_NODES, HIDDEN, HIDDEN)
    b2r = b2.reshape(N_NODES, HIDDEN)

    eye = jnp.eye(HEADS, dtype=jnp.float32)
    p_src = (eye[:, None, :] * att_src[:, :, None]).reshape(HIDDEN, HEADS)
    p_dst = (eye[:, None, :] * att_dst[:, :, None]).reshape(HIDDEN, HEADS)
    P = jnp.concatenate([p_src, p_dst], axis=1)

    g = -jnp.log(-jnp.log(_GUMBEL_U))

    const = lambda i: (0, 0)
    adj = pl.pallas_call(
        _fused_kernel,
        grid=(NB + B // GPB,),
        in_specs=[
            pl.BlockSpec((B, LATENT), const),
            pl.BlockSpec((HIDDEN, LATENT), const),
            pl.BlockSpec((1, HIDDEN), const),
            pl.BlockSpec((HIDDEN, HIDDEN), const),
            pl.BlockSpec((1, HIDDEN), const),
            pl.BlockSpec((NODE_BLK, HIDDEN, HIDDEN),
                         lambda i: (jnp.minimum(i, NB - 1), 0, 0)),
            pl.BlockSpec((NODE_BLK, HIDDEN),
                         lambda i: (jnp.minimum(i, NB - 1), 0)),
            pl.BlockSpec((HIDDEN, HIDDEN), const),
            pl.BlockSpec((HIDDEN, 2 * HEADS), const),
            pl.BlockSpec((1, HIDDEN), const),
            pl.BlockSpec((N_NODES, HIDDEN), const),
            pl.BlockSpec((1, N_NODES), const),
            pl.BlockSpec((GPB, N_NODES, N_NODES),
                         lambda i: (jnp.maximum(i - NB, 0), 0, 0)),
        ],
        out_specs=pl.BlockSpec((GPB, N_NODES, N_NODES),
                               lambda i: (jnp.maximum(i - NB, 0), 0, 0)),
        out_shape=jax.ShapeDtypeStruct((B, N_NODES, N_NODES), jnp.float32),
        scratch_shapes=[pltpu.VMEM((NB, B, NODE_BLK, HIDDEN), jnp.float32)],
    )(x, W0, b0.reshape(1, -1), W1, b1.reshape(1, -1), w2r, b2r, gat_W,
      P, gat_b.reshape(1, -1), Wl, bl.reshape(1, -1), g)
    return adj


# speedup vs baseline: 1.2064x; 1.0142x over previous
import jax
import jax.numpy as jnp
import numpy as np
from jax.experimental import pallas as pl
from jax.experimental.pallas import tpu as pltpu

LATENT = 128
HIDDEN = 128
N_NODES = 256
HEADS = 4
OUTC = HIDDEN // HEADS
BATCH = 16
NEG_SLOPE = 0.2

NODE_BLK = 32
NB = N_NODES // NODE_BLK
GPB = 4
_PREC = jax.lax.Precision.HIGHEST


def _np_threefry_uniform(seed, shape):
    def rotl(x, d):
        return (x << np.uint32(d)) | (x >> np.uint32(32 - d))

    n = int(np.prod(shape))
    idx = np.arange(n, dtype=np.uint64)
    c1 = (idx >> np.uint64(32)).astype(np.uint32)
    c2 = (idx & np.uint64(0xFFFFFFFF)).astype(np.uint32)
    k1 = np.uint32(seed >> 32)
    k2 = np.uint32(seed & 0xFFFFFFFF)
    rot = ([13, 15, 26, 6], [17, 29, 16, 24])
    ks = (k1, k2, k1 ^ k2 ^ np.uint32(0x1BD11BDA))
    x = [c1 + ks[0], c2 + ks[1]]
    for i in range(5):
        for r in rot[i % 2]:
            x[0] = x[0] + x[1]
            x[1] = rotl(x[1], r) ^ x[0]
        x[0] = x[0] + ks[(i + 1) % 3]
        x[1] = x[1] + ks[(i + 2) % 3] + np.uint32(i + 1)
    bits = x[0] ^ x[1]
    float_bits = (bits >> np.uint32(9)) | np.uint32(0x3F800000)
    floats = float_bits.view(np.float32) - np.float32(1.0)
    tiny = np.float32(np.finfo(np.float32).tiny)
    span = np.float32(np.float32(1.0) - tiny)
    return np.maximum(tiny, floats * span + tiny).reshape(shape)


_GUMBEL_U = _np_threefry_uniform(42, (BATCH, N_NODES, N_NODES))


def _fused_kernel(x_ref, w0_ref, b0_ref, w1_ref, b1_ref, w2_ref, b2_ref,
                  gw_ref, p_ref, gb_ref, wl_ref, bl_ref, g_ref,
                  out_ref, xh_scr):
    i = pl.program_id(0)

    @pl.when(i < NB)
    def _stage_a():
        h = jax.lax.dot_general(x_ref[...], w0_ref[...],
                                (((1,), (1,)), ((), ())), precision=_PREC)
        h = jnp.maximum(h + b0_ref[...], 0.0)
        h = jax.lax.dot_general(h, w1_ref[...], (((1,), (1,)), ((), ())),
                                precision=_PREC)
        h = jnp.maximum(h + b1_ref[...], 0.0)
        h2 = jax.lax.dot_general(h, w2_ref[...], (((1,), (2,)), ((), ())),
                                 precision=_PREC)
        h2 = h2 + b2_ref[...][None, :, :]
        xh = jax.lax.dot_general(h2, gw_ref[...], (((2,), (1,)), ((), ())),
                                 precision=_PREC)
        xh_scr[i] = xh

    @pl.when(i >= NB)
    def _stage_b():
        for gi in range(GPB):
            _one_graph(i - NB, gi, p_ref, gb_ref, wl_ref, bl_ref, g_ref,
                       out_ref, xh_scr)


def _one_graph(b, gi, p_ref, gb_ref, wl_ref, bl_ref, g_ref, out_ref, xh_scr):
    if True:
        xh = jnp.concatenate([xh_scr[nb, b * GPB + gi] for nb in range(NB)],
                             axis=0)
        sc = jnp.dot(xh, p_ref[...], precision=_PREC)
        scT = jax.lax.dot_general(p_ref[...], xh, (((0,), (1,)), ((), ())),
                                  precision=_PREC)
        HN = N_NODES // 2
        ii = jax.lax.broadcasted_iota(jnp.int32, (HN, HN), 0)
        jj = jax.lax.broadcasted_iota(jnp.int32, (HN, HN), 1)
        tri = jnp.where(ii <= jj, 0.0, -jnp.inf)
        mask_lo = jnp.concatenate(
            [tri, jnp.zeros((HN, HN), jnp.float32)], axis=1)
        ones_col = jnp.ones((N_NODES, 1), dtype=jnp.float32)
        aggs = []
        for h in range(HEADS):
            arow = scT[HEADS + h:HEADS + h + 1, :]
            sl = sc[:HN, h:h + 1] + arow
            sh = sc[HN:, h:h + 1] + arow[:, HN:]
            sl = jnp.where(sl >= 0.0, sl, NEG_SLOPE * sl) + mask_lo
            sh = jnp.where(sh >= 0.0, sh, NEG_SLOPE * sh) + tri
            cl = jnp.max(sl, axis=0, keepdims=True)
            ch = jnp.max(sh, axis=0, keepdims=True)
            cmax = jnp.concatenate(
                [cl[:, :HN], jnp.maximum(cl[:, HN:], ch)], axis=1)
            ex_lo = jnp.exp(sl - cmax)
            ex_hi = jnp.exp(sh - cmax[:, HN:])
            xh_h = jnp.concatenate(
                [xh[:, h * OUTC:(h + 1) * OUTC], ones_col],
                axis=1)
            m = jax.lax.dot_general(ex_lo, xh_h[:HN],
                                    (((0,), (0,)), ((), ())),
                                    precision=_PREC)
            mb = jax.lax.dot_general(ex_hi, xh_h[HN:],
                                     (((0,), (0,)), ((), ())),
                                     precision=_PREC)
            m = m + jnp.concatenate(
                [jnp.zeros((HN, OUTC + 1), jnp.float32), mb], axis=0)
            recip = 1.0 / (m[:, OUTC:OUTC + 1] + 1e-16)
            aggs.append(m[:, :OUTC] * recip)
        agg = jnp.concatenate(aggs, axis=1)
        out = agg + gb_ref[...]
        out = jnp.where(out > 0.0, out,
                        jnp.exp(jnp.minimum(out, 0.0)) - 1.0)
        logits = jax.lax.dot_general(out, wl_ref[...],
                                     (((1,), (1,)), ((), ())),
                                     precision=_PREC)
        z = logits + bl_ref[...] + g_ref[gi]
        jjf = jax.lax.broadcasted_iota(jnp.int32, (N_NODES, N_NODES), 1)
        rmax = jnp.max(z, axis=1, keepdims=True)
        eq = z == rmax
        idx = jnp.min(jnp.where(eq, jjf, N_NODES), axis=1, keepdims=True)
        y = (jjf == idx).astype(jnp.float32)
        adj = jnp.minimum(y + y.T, 1.0)
        out_ref[gi] = adj


@jax.jit
def kernel(x, W0, b0, W1, b1, W2, b2, gat_W, att_src, att_dst, gat_b, Wl, bl):
    B = x.shape[0]
    w2r = W2.reshape(N_NODES, HIDDEN, HIDDEN)
    b2r = b2.reshape(N_NODES, HIDDEN)

    eye = jnp.eye(HEADS, dtype=jnp.float32)
    p_src = (eye[:, None, :] * att_src[:, :, None]).reshape(HIDDEN, HEADS)
    p_dst = (eye[:, None, :] * att_dst[:, :, None]).reshape(HIDDEN, HEADS)
    P = jnp.concatenate([p_src, p_dst], axis=1)

    g = -jnp.log(-jnp.log(_GUMBEL_U))

    const = lambda i: (0, 0)
    adj = pl.pallas_call(
        _fused_kernel,
        grid=(NB + B // GPB,),
        in_specs=[
            pl.BlockSpec((B, LATENT), const),
            pl.BlockSpec((HIDDEN, LATENT), const),
            pl.BlockSpec((1, HIDDEN), const),
            pl.BlockSpec((HIDDEN, HIDDEN), const),
            pl.BlockSpec((1, HIDDEN), const),
            pl.BlockSpec((NODE_BLK, HIDDEN, HIDDEN),
                         lambda i: (jnp.minimum(i, NB - 1), 0, 0)),
            pl.BlockSpec((NODE_BLK, HIDDEN),
                         lambda i: (jnp.minimum(i, NB - 1), 0)),
            pl.BlockSpec((HIDDEN, HIDDEN), const),
            pl.BlockSpec((HIDDEN, 2 * HEADS), const),
            pl.BlockSpec((1, HIDDEN), const),
            pl.BlockSpec((N_NODES, HIDDEN), const),
            pl.BlockSpec((1, N_NODES), const),
            pl.BlockSpec((GPB, N_NODES, N_NODES),
                         lambda i: (jnp.maximum(i - NB, 0), 0, 0)),
        ],
        out_specs=pl.BlockSpec((GPB, N_NODES, N_NODES),
                               lambda i: (jnp.maximum(i - NB, 0), 0, 0)),
        out_shape=jax.ShapeDtypeStruct((B, N_NODES, N_NODES), jnp.float32),
        scratch_shapes=[pltpu.VMEM((NB, B, NODE_BLK, HIDDEN), jnp.float32)],
    )(x, W0, b0.reshape(1, -1), W1, b1.reshape(1, -1), w2r, b2r, gat_W,
      P, gat_b.reshape(1, -1), Wl, bl.reshape(1, -1), g)
    return adj


# speedup vs baseline: 1.2438x; 1.0310x over previous
import jax
import jax.numpy as jnp
import numpy as np
from jax.experimental import pallas as pl
from jax.experimental.pallas import tpu as pltpu

LATENT = 128
HIDDEN = 128
N_NODES = 256
HEADS = 4
OUTC = HIDDEN // HEADS
BATCH = 16
NEG_SLOPE = 0.2

NODE_BLK = 32
NB = N_NODES // NODE_BLK
GPB = 8
_PREC = jax.lax.Precision.HIGHEST


def _np_threefry_uniform(seed, shape):
    def rotl(x, d):
        return (x << np.uint32(d)) | (x >> np.uint32(32 - d))

    n = int(np.prod(shape))
    idx = np.arange(n, dtype=np.uint64)
    c1 = (idx >> np.uint64(32)).astype(np.uint32)
    c2 = (idx & np.uint64(0xFFFFFFFF)).astype(np.uint32)
    k1 = np.uint32(seed >> 32)
    k2 = np.uint32(seed & 0xFFFFFFFF)
    rot = ([13, 15, 26, 6], [17, 29, 16, 24])
    ks = (k1, k2, k1 ^ k2 ^ np.uint32(0x1BD11BDA))
    x = [c1 + ks[0], c2 + ks[1]]
    for i in range(5):
        for r in rot[i % 2]:
            x[0] = x[0] + x[1]
            x[1] = rotl(x[1], r) ^ x[0]
        x[0] = x[0] + ks[(i + 1) % 3]
        x[1] = x[1] + ks[(i + 2) % 3] + np.uint32(i + 1)
    bits = x[0] ^ x[1]
    float_bits = (bits >> np.uint32(9)) | np.uint32(0x3F800000)
    floats = float_bits.view(np.float32) - np.float32(1.0)
    tiny = np.float32(np.finfo(np.float32).tiny)
    span = np.float32(np.float32(1.0) - tiny)
    return np.maximum(tiny, floats * span + tiny).reshape(shape)


_GUMBEL_U = _np_threefry_uniform(42, (BATCH, N_NODES, N_NODES))


def _fused_kernel(x_ref, w0_ref, b0_ref, w1_ref, b1_ref, w2_ref, b2_ref,
                  gw_ref, p_ref, gb_ref, wl_ref, bl_ref, g_ref,
                  out_ref, xh_scr):
    i = pl.program_id(0)

    @pl.when(i < NB)
    def _stage_a():
        h = jax.lax.dot_general(x_ref[...], w0_ref[...],
                                (((1,), (1,)), ((), ())), precision=_PREC)
        h = jnp.maximum(h + b0_ref[...], 0.0)
        h = jax.lax.dot_general(h, w1_ref[...], (((1,), (1,)), ((), ())),
                                precision=_PREC)
        h = jnp.maximum(h + b1_ref[...], 0.0)
        h2 = jax.lax.dot_general(h, w2_ref[...], (((1,), (2,)), ((), ())),
                                 precision=_PREC)
        h2 = h2 + b2_ref[...][None, :, :]
        xh = jax.lax.dot_general(h2, gw_ref[...], (((2,), (1,)), ((), ())),
                                 precision=_PREC)
        xh_scr[i] = xh

    @pl.when(i >= NB)
    def _stage_b():
        for gi in range(GPB):
            _one_graph(i - NB, gi, p_ref, gb_ref, wl_ref, bl_ref, g_ref,
                       out_ref, xh_scr)


def _one_graph(b, gi, p_ref, gb_ref, wl_ref, bl_ref, g_ref, out_ref, xh_scr):
    if True:
        xh = jnp.concatenate([xh_scr[nb, b * GPB + gi] for nb in range(NB)],
                             axis=0)
        sc = jnp.dot(xh, p_ref[...], precision=_PREC)
        scT = jax.lax.dot_general(p_ref[...], xh, (((0,), (1,)), ((), ())),
                                  precision=_PREC)
        HN = N_NODES // 2
        ii = jax.lax.broadcasted_iota(jnp.int32, (HN, HN), 0)
        jj = jax.lax.broadcasted_iota(jnp.int32, (HN, HN), 1)
        tri = jnp.where(ii <= jj, 0.0, -jnp.inf)
        mask_lo = jnp.concatenate(
            [tri, jnp.zeros((HN, HN), jnp.float32)], axis=1)
        ones_col = jnp.ones((N_NODES, 1), dtype=jnp.float32)
        aggs = []
        for h in range(HEADS):
            arow = scT[HEADS + h:HEADS + h + 1, :]
            sl = sc[:HN, h:h + 1] + arow
            sh = sc[HN:, h:h + 1] + arow[:, HN:]
            sl = jnp.where(sl >= 0.0, sl, NEG_SLOPE * sl) + mask_lo
            sh = jnp.where(sh >= 0.0, sh, NEG_SLOPE * sh) + tri
            cl = jnp.max(sl, axis=0, keepdims=True)
            ch = jnp.max(sh, axis=0, keepdims=True)
            cmax = jnp.concatenate(
                [cl[:, :HN], jnp.maximum(cl[:, HN:], ch)], axis=1)
            ex_lo = jnp.exp(sl - cmax)
            ex_hi = jnp.exp(sh - cmax[:, HN:])
            xh_h = jnp.concatenate(
                [xh[:, h * OUTC:(h + 1) * OUTC], ones_col],
                axis=1)
            m = jax.lax.dot_general(ex_lo, xh_h[:HN],
                                    (((0,), (0,)), ((), ())),
                                    precision=_PREC)
            mb = jax.lax.dot_general(ex_hi, xh_h[HN:],
                                     (((0,), (0,)), ((), ())),
                                     precision=_PREC)
            m = m + jnp.concatenate(
                [jnp.zeros((HN, OUTC + 1), jnp.float32), mb], axis=0)
            recip = 1.0 / (m[:, OUTC:OUTC + 1] + 1e-16)
            aggs.append(m[:, :OUTC] * recip)
        agg = jnp.concatenate(aggs, axis=1)
        out = agg + gb_ref[...]
        out = jnp.where(out > 0.0, out,
                        jnp.exp(jnp.minimum(out, 0.0)) - 1.0)
        logits = jax.lax.dot_general(out, wl_ref[...],
                                     (((1,), (1,)), ((), ())),
                                     precision=_PREC)
        z = logits + bl_ref[...] + g_ref[gi]
        jjf = jax.lax.broadcasted_iota(jnp.int32, (N_NODES, N_NODES), 1)
        rmax = jnp.max(z, axis=1, keepdims=True)
        eq = z == rmax
        idx = jnp.min(jnp.where(eq, jjf, N_NODES), axis=1, keepdims=True)
        y = (jjf == idx).astype(jnp.float32)
        adj = jnp.minimum(y + y.T, 1.0)
        out_ref[gi] = adj


@jax.jit
def kernel(x, W0, b0, W1, b1, W2, b2, gat_W, att_src, att_dst, gat_b, Wl, bl):
    B = x.shape[0]
    w2r = W2.reshape(N_NODES, HIDDEN, HIDDEN)
    b2r = b2.reshape(N_NODES, HIDDEN)

    eye = jnp.eye(HEADS, dtype=jnp.float32)
    p_src = (eye[:, None, :] * att_src[:, :, None]).reshape(HIDDEN, HEADS)
    p_dst = (eye[:, None, :] * att_dst[:, :, None]).reshape(HIDDEN, HEADS)
    P = jnp.concatenate([p_src, p_dst], axis=1)

    g = -jnp.log(-jnp.log(_GUMBEL_U))

    const = lambda i: (0, 0)
    adj = pl.pallas_call(
        _fused_kernel,
        grid=(NB + B // GPB,),
        in_specs=[
            pl.BlockSpec((B, LATENT), const),
            pl.BlockSpec((HIDDEN, LATENT), const),
            pl.BlockSpec((1, HIDDEN), const),
            pl.BlockSpec((HIDDEN, HIDDEN), const),
            pl.BlockSpec((1, HIDDEN), const),
            pl.BlockSpec((NODE_BLK, HIDDEN, HIDDEN),
                         lambda i: (jnp.minimum(i, NB - 1), 0, 0)),
            pl.BlockSpec((NODE_BLK, HIDDEN),
                         lambda i: (jnp.minimum(i, NB - 1), 0)),
            pl.BlockSpec((HIDDEN, HIDDEN), const),
            pl.BlockSpec((HIDDEN, 2 * HEADS), const),
            pl.BlockSpec((1, HIDDEN), const),
            pl.BlockSpec((N_NODES, HIDDEN), const),
            pl.BlockSpec((1, N_NODES), const),
            pl.BlockSpec((GPB, N_NODES, N_NODES),
                         lambda i: (jnp.maximum(i - NB, 0), 0, 0)),
        ],
        out_specs=pl.BlockSpec((GPB, N_NODES, N_NODES),
                               lambda i: (jnp.maximum(i - NB, 0), 0, 0)),
        out_shape=jax.ShapeDtypeStruct((B, N_NODES, N_NODES), jnp.float32),
        scratch_shapes=[pltpu.VMEM((NB, B, NODE_BLK, HIDDEN), jnp.float32)],
    )(x, W0, b0.reshape(1, -1), W1, b1.reshape(1, -1), w2r, b2r, gat_W,
      P, gat_b.reshape(1, -1), Wl, bl.reshape(1, -1), g)
    return adj
